# Initial kernel scaffold; baseline (speedup 1.0000x reference)
import jax
import jax.numpy as jnp
from jax.experimental import pallas as pl


def kernel(class_indices, templates):
    raise NotImplementedError("write your pallas kernel here")


# speedup vs baseline: 4.4110x; 4.4110x over previous
import functools

import jax
import jax.numpy as jnp
from jax import lax
from jax.experimental import pallas as pl
from jax.experimental.pallas import tpu as pltpu
from jax.experimental.pallas import tpu_sc as plsc

L = 16
CHUNK = 2048


def kernel(class_indices, templates):
    B, C, H, W = class_indices.shape
    K, R, S = templates.shape
    D = R * S
    N = B * C * H * W

    idx_flat = class_indices.reshape(N).astype(jnp.int32)
    tab_flat = templates.reshape(K * D)

    info = plsc.get_sparse_core_info()
    nw = info.num_cores * info.num_subcores
    per_w = N // nw
    assert per_w * nw == N and per_w % CHUNK == 0
    chunks = per_w // CHUNK
    groups = CHUNK // L

    mesh = plsc.VectorSubcoreMesh(core_axis_name="c", subcore_axis_name="s")

    @functools.partial(
        pl.kernel,
        mesh=mesh,
        compiler_params=pltpu.CompilerParams(needs_layout_passes=False),
        out_type=jax.ShapeDtypeStruct((N * D,), jnp.float32),
        scratch_types=[
            pltpu.VMEM((K * D,), jnp.float32),
            pltpu.VMEM((CHUNK,), jnp.int32),
            pltpu.VMEM((CHUNK * D,), jnp.float32),
        ],
    )
    def sc_lookup(idx_hbm, tab_hbm, out_hbm, tab_v, idx_v, out_v):
        wid = lax.axis_index("s") * info.num_cores + lax.axis_index("c")
        base = wid * per_w
        pltpu.sync_copy(tab_hbm, tab_v)

        lane = lax.iota(jnp.int32, L)
        n_offs, j_offs = [], []
        for sub in range(D):
            t = sub * L + lane
            n_off = lax.div(t, jnp.int32(D))
            n_offs.append(n_off)
            j_offs.append(t - D * n_off)

        def chunk_body(c, carry):
            off = base + c * CHUNK
            pltpu.sync_copy(idx_hbm.at[pl.ds(off, CHUNK)], idx_v)

            def group_body(g, carry2):
                gbase = g * L
                obase = g * (L * D)
                for sub in range(D):
                    ids = plsc.load_gather(idx_v, [gbase + n_offs[sub]])
                    val = plsc.load_gather(tab_v, [ids * D + j_offs[sub]])
                    out_v[pl.ds(obase + sub * L, L)] = val
                return carry2

            lax.fori_loop(0, groups, group_body, 0, unroll=False)
            pltpu.sync_copy(out_v, out_hbm.at[pl.ds(off * D, CHUNK * D)])
            return carry

        lax.fori_loop(0, chunks, chunk_body, 0, unroll=False)

    out = sc_lookup(idx_flat, tab_flat)
    return out.reshape(B, C, H, W, R, S)


# speedup vs baseline: 4.8726x; 1.1046x over previous
import functools

import jax
import jax.numpy as jnp
from jax import lax
from jax.experimental import pallas as pl
from jax.experimental.pallas import tpu as pltpu
from jax.experimental.pallas import tpu_sc as plsc

L = 16
CHUNK = 2048


def kernel(class_indices, templates):
    B, C, H, W = class_indices.shape
    K, R, S = templates.shape
    D = R * S
    N = B * C * H * W

    idx_flat = class_indices.reshape(N).astype(jnp.int32)
    tab_flat = templates.reshape(K * D)

    info = plsc.get_sparse_core_info()
    nw = info.num_cores * info.num_subcores
    per_w = N // nw
    assert per_w * nw == N and per_w % CHUNK == 0
    chunks = per_w // CHUNK
    groups = CHUNK // L

    mesh = plsc.VectorSubcoreMesh(core_axis_name="c", subcore_axis_name="s")

    @functools.partial(
        pl.kernel,
        mesh=mesh,
        compiler_params=pltpu.CompilerParams(needs_layout_passes=False),
        out_type=jax.ShapeDtypeStruct((N * D,), jnp.float32),
        scratch_types=[
            pltpu.VMEM((K * D,), jnp.float32),
            pltpu.VMEM((CHUNK,), jnp.int32),
            pltpu.VMEM((CHUNK * D,), jnp.float32),
        ],
    )
    def sc_lookup(idx_hbm, tab_hbm, out_hbm, tab_v, idx_v, out_v):
        wid = lax.axis_index("s") * info.num_cores + lax.axis_index("c")
        base = wid * per_w
        pltpu.sync_copy(tab_hbm, tab_v)

        lane = lax.iota(jnp.int32, L)
        n_offs, j_offs = [], []
        for sub in range(D):
            t = sub * L + lane
            n_off = lax.div(t, jnp.int32(D))
            n_offs.append(n_off)
            j_offs.append(t - D * n_off)

        def chunk_body(c, carry):
            off = base + c * CHUNK
            pltpu.sync_copy(idx_hbm.at[pl.ds(off, CHUNK)], idx_v)

            @plsc.parallel_loop(0, groups, step=1, unroll=4)
            def group_body(g):
                gbase = g * L
                obase = g * (L * D)
                for sub in range(D):
                    ids = plsc.load_gather(idx_v, [gbase + n_offs[sub]])
                    val = plsc.load_gather(tab_v, [ids * D + j_offs[sub]])
                    out_v[pl.ds(obase + sub * L, L)] = val
            pltpu.sync_copy(out_v, out_hbm.at[pl.ds(off * D, CHUNK * D)])
            return carry

        lax.fori_loop(0, chunks, chunk_body, 0, unroll=False)

    out = sc_lookup(idx_flat, tab_flat)
    return out.reshape(B, C, H, W, R, S)


# speedup vs baseline: 156.7092x; 32.1615x over previous
import functools

import jax
import jax.numpy as jnp
from jax import lax
from jax.experimental import pallas as pl
from jax.experimental.pallas import tpu as pltpu
from jax.experimental.pallas import tpu_sc as plsc

L = 16
HS = 32


def kernel(class_indices, templates):
    B, C, H, W = class_indices.shape
    K, R, S = templates.shape
    D = R * S
    N = B * C * H * W
    BC = B * C
    task_idx = HS * W

    idx_flat = class_indices.reshape(N).astype(jnp.int32)
    tab_flat = templates.reshape(K * D)

    info = plsc.get_sparse_core_info()
    nw = info.num_cores * info.num_subcores
    hslices = H // HS
    tasks = BC * hslices
    per_w = tasks // nw
    assert per_w * nw == tasks and HS * W % L == 0
    groups = task_idx // L

    mesh = plsc.VectorSubcoreMesh(core_axis_name="c", subcore_axis_name="s")

    @functools.partial(
        pl.kernel,
        mesh=mesh,
        compiler_params=pltpu.CompilerParams(needs_layout_passes=False),
        out_type=jax.ShapeDtypeStruct((N * D,), jnp.float32),
        scratch_types=[
            pltpu.VMEM((K * D,), jnp.float32),
            pltpu.VMEM((task_idx,), jnp.int32),
            pltpu.VMEM((D * task_idx,), jnp.float32),
        ],
    )
    def sc_lookup(idx_hbm, tab_hbm, out_hbm, tab_v, idx_v, out_v):
        wid = lax.axis_index("s") * info.num_cores + lax.axis_index("c")
        base_task = wid * per_w
        pltpu.sync_copy(tab_hbm, tab_v)

        def task_body(t, carry):
            task = base_task + t
            bc = lax.div(task, jnp.int32(hslices))
            hs = lax.rem(task, jnp.int32(hslices))
            in_off = bc * (H * W) + hs * task_idx
            pltpu.sync_copy(idx_hbm.at[pl.ds(in_off, task_idx)], idx_v)

            @plsc.parallel_loop(0, groups, step=1, unroll=4)
            def group_body(g):
                gbase = g * L
                ids = idx_v[pl.ds(gbase, L)] * D
                for j in range(D):
                    out_v[pl.ds(j * task_idx + gbase, L)] = plsc.load_gather(
                        tab_v, [ids + j]
                    )

            for j in range(D):
                pltpu.sync_copy(
                    out_v.at[pl.ds(j * task_idx, task_idx)],
                    out_hbm.at[pl.ds((bc * D + j) * (H * W) + hs * task_idx, task_idx)],
                )
            return carry

        lax.fori_loop(0, per_w, task_body, 0, unroll=False)

    out = sc_lookup(idx_flat, tab_flat)
    out = out.reshape(B, C, R, S, H, W)
    return out.transpose(0, 1, 4, 5, 2, 3)
